# baseline (device time: 429428 ns/iter reference)
import jax
import jax.numpy as jnp
from jax import lax
from jax.experimental import pallas as pl
from jax.experimental.pallas import tpu as pltpu

N_DEV = 32
AXIS = "i"


def kernel(x, w_mat):
    m, k_shard = x.shape
    _, n = w_mat.shape
    blk = m // N_DEV

    def body(x_ref, w_ref, out_ref, comm_ref, acc_ref,
             send_sems, recv_sems,
             amax_send_ref, amax_recv_ref, amax_send_sems, amax_recv_sems):
        p = lax.axis_index(AXIS)
        right = lax.rem(p + 1, N_DEV)
        left = lax.rem(p + N_DEV - 1, N_DEV)

        barrier_sem = pltpu.get_barrier_semaphore()
        pl.semaphore_signal(barrier_sem, inc=1, device_id=(left,),
                            device_id_type=pl.DeviceIdType.MESH)
        pl.semaphore_signal(barrier_sem, inc=1, device_id=(right,),
                            device_id_type=pl.DeviceIdType.MESH)
        pl.semaphore_wait(barrier_sem, 2)

        def contrib(b):
            xb = x_ref[pl.ds(b * blk, blk), :]
            return jnp.dot(xb, w_ref[...], preferred_element_type=jnp.float32)

        comm_ref[0] = contrib(lax.rem(p + N_DEV - 1, N_DEV))
        for s in range(N_DEV - 1):
            send_slot = s % 2
            recv_slot = (s + 1) % 2
            rdma = pltpu.make_async_remote_copy(
                src_ref=comm_ref.at[send_slot],
                dst_ref=comm_ref.at[recv_slot],
                send_sem=send_sems.at[send_slot],
                recv_sem=recv_sems.at[recv_slot],
                device_id=(right,),
                device_id_type=pl.DeviceIdType.MESH,
            )
            rdma.start()
            nxt = contrib(lax.rem(p + 2 * N_DEV - s - 2, N_DEV))
            rdma.wait()
            if s < N_DEV - 2:
                comm_ref[recv_slot] = comm_ref[recv_slot] + nxt
            else:
                acc_ref[...] = comm_ref[recv_slot] + nxt

        y = acc_ref[...]
        amax = jnp.max(jnp.abs(y))
        for k in range(5):
            partner = p ^ (1 << k)
            amax_send_ref[...] = jnp.full((8, 128), amax, jnp.float32)
            ex = pltpu.make_async_remote_copy(
                src_ref=amax_send_ref,
                dst_ref=amax_recv_ref.at[k],
                send_sem=amax_send_sems.at[k],
                recv_sem=amax_recv_sems.at[k],
                device_id=(partner,),
                device_id_type=pl.DeviceIdType.MESH,
            )
            ex.start()
            ex.wait()
            amax = jnp.maximum(amax, amax_recv_ref[k, 0, 0])

        scale = amax / 127.0
        q = jnp.clip(jnp.round(y / scale), -127.0, 127.0)
        out_ref[...] = q * scale

    return pl.pallas_call(
        body,
        out_shape=jax.ShapeDtypeStruct((blk, n), jnp.float32),
        in_specs=[
            pl.BlockSpec(memory_space=pltpu.VMEM),
            pl.BlockSpec(memory_space=pltpu.VMEM),
        ],
        out_specs=pl.BlockSpec(memory_space=pltpu.VMEM),
        scratch_shapes=[
            pltpu.VMEM((2, blk, n), jnp.float32),
            pltpu.VMEM((blk, n), jnp.float32),
            pltpu.SemaphoreType.DMA((2,)),
            pltpu.SemaphoreType.DMA((2,)),
            pltpu.VMEM((8, 128), jnp.float32),
            pltpu.VMEM((5, 8, 128), jnp.float32),
            pltpu.SemaphoreType.DMA((5,)),
            pltpu.SemaphoreType.DMA((5,)),
        ],
        compiler_params=pltpu.CompilerParams(collective_id=0),
    )(x, w_mat)


# device time: 249409 ns/iter; 1.7218x vs baseline; 1.7218x over previous
import jax
import jax.numpy as jnp
from jax import lax
from jax.experimental import pallas as pl
from jax.experimental.pallas import tpu as pltpu

N_DEV = 32
AXIS = "i"


def _coords_of_logical(p):
    z = p // 8
    r8 = lax.rem(p, 8)
    y = r8 // 2
    j = lax.rem(r8, 2)
    x = jnp.where(lax.rem(y, 2) == 0, j, 1 - j)
    return x, y, z


def _logical_of_coords(x, y, z):
    xl = jnp.where(lax.rem(y, 2) == 0, x, 1 - x)
    return z * 8 + y * 2 + xl


def _rank_of_logical(p):
    x, y, z = _coords_of_logical(p)
    k = z * 4 + jnp.where(lax.rem(z, 2) == 0, y, 3 - y)
    return jnp.where(x == 0, k, 31 - k)


def _logical_of_rank(r):
    r = lax.rem(r + 2 * N_DEV, N_DEV)
    k = jnp.where(r < 16, r, 31 - r)
    x = jnp.where(r < 16, 0, 1)
    z = k // 4
    y4 = lax.rem(k, 4)
    y = jnp.where(lax.rem(z, 2) == 0, y4, 3 - y4)
    return _logical_of_coords(x, y, z)


def kernel(x, w_mat):
    m, k_shard = x.shape
    _, n = w_mat.shape
    blk = m // N_DEV
    nh = n // 2

    def body(x_ref, w_ref, out_ref, comm_r, comm_l, acc_ref,
             send_sems_r, recv_sems_r, send_sems_l, recv_sems_l,
             amax_send_ref, amax_recv_ref, amax_send_sems, amax_recv_sems):
        p = lax.axis_index(AXIS)
        rk = _rank_of_logical(p)
        right = _logical_of_rank(rk + 1)
        left = _logical_of_rank(rk - 1)

        barrier_sem = pltpu.get_barrier_semaphore()
        pl.semaphore_signal(barrier_sem, inc=1, device_id=(left,),
                            device_id_type=pl.DeviceIdType.MESH)
        pl.semaphore_signal(barrier_sem, inc=1, device_id=(right,),
                            device_id_type=pl.DeviceIdType.MESH)
        pl.semaphore_wait(barrier_sem, 2)

        def contrib_r(dest_rank):
            b = _logical_of_rank(dest_rank)
            xb = x_ref[pl.ds(b * blk, blk), :]
            return jnp.dot(xb, w_ref[:, :nh], preferred_element_type=jnp.float32)

        def contrib_l(dest_rank):
            b = _logical_of_rank(dest_rank)
            xb = x_ref[pl.ds(b * blk, blk), :]
            return jnp.dot(xb, w_ref[:, nh:], preferred_element_type=jnp.float32)

        comm_r[0] = contrib_r(rk - 1)
        comm_l[0] = contrib_l(rk + 1)
        for s in range(N_DEV - 1):
            send_slot = s % 2
            recv_slot = (s + 1) % 2
            rdma_r = pltpu.make_async_remote_copy(
                src_ref=comm_r.at[send_slot],
                dst_ref=comm_r.at[recv_slot],
                send_sem=send_sems_r.at[send_slot],
                recv_sem=recv_sems_r.at[recv_slot],
                device_id=(right,),
                device_id_type=pl.DeviceIdType.MESH,
            )
            rdma_l = pltpu.make_async_remote_copy(
                src_ref=comm_l.at[send_slot],
                dst_ref=comm_l.at[recv_slot],
                send_sem=send_sems_l.at[send_slot],
                recv_sem=recv_sems_l.at[recv_slot],
                device_id=(left,),
                device_id_type=pl.DeviceIdType.MESH,
            )
            rdma_r.start()
            rdma_l.start()
            nxt_r = contrib_r(rk - s - 2)
            nxt_l = contrib_l(rk + s + 2)
            rdma_r.wait()
            rdma_l.wait()
            if s < N_DEV - 2:
                comm_r[recv_slot] = comm_r[recv_slot] + nxt_r
                comm_l[recv_slot] = comm_l[recv_slot] + nxt_l
            else:
                acc_ref[:, :nh] = comm_r[recv_slot] + nxt_r
                acc_ref[:, nh:] = comm_l[recv_slot] + nxt_l

        y = acc_ref[...]
        amax = jnp.max(jnp.abs(y))
        for k in range(5):
            partner = p ^ (1 << k)
            amax_send_ref[...] = jnp.full((8, 128), amax, jnp.float32)
            ex = pltpu.make_async_remote_copy(
                src_ref=amax_send_ref,
                dst_ref=amax_recv_ref.at[k],
                send_sem=amax_send_sems.at[k],
                recv_sem=amax_recv_sems.at[k],
                device_id=(partner,),
                device_id_type=pl.DeviceIdType.MESH,
            )
            ex.start()
            ex.wait()
            amax = jnp.maximum(amax, amax_recv_ref[k, 0, 0])

        scale = amax / 127.0
        q = jnp.clip(jnp.round(y / scale), -127.0, 127.0)
        out_ref[...] = q * scale

    return pl.pallas_call(
        body,
        out_shape=jax.ShapeDtypeStruct((blk, n), jnp.float32),
        in_specs=[
            pl.BlockSpec(memory_space=pltpu.VMEM),
            pl.BlockSpec(memory_space=pltpu.VMEM),
        ],
        out_specs=pl.BlockSpec(memory_space=pltpu.VMEM),
        scratch_shapes=[
            pltpu.VMEM((2, blk, nh), jnp.float32),
            pltpu.VMEM((2, blk, nh), jnp.float32),
            pltpu.VMEM((blk, n), jnp.float32),
            pltpu.SemaphoreType.DMA((2,)),
            pltpu.SemaphoreType.DMA((2,)),
            pltpu.SemaphoreType.DMA((2,)),
            pltpu.SemaphoreType.DMA((2,)),
            pltpu.VMEM((8, 128), jnp.float32),
            pltpu.VMEM((5, 8, 128), jnp.float32),
            pltpu.SemaphoreType.DMA((5,)),
            pltpu.SemaphoreType.DMA((5,)),
        ],
        compiler_params=pltpu.CompilerParams(collective_id=0),
    )(x, w_mat)


# device time: 191483 ns/iter; 2.2426x vs baseline; 1.3025x over previous
import jax
import jax.numpy as jnp
from jax import lax
from jax.experimental import pallas as pl
from jax.experimental.pallas import tpu as pltpu

N_DEV = 32
AXIS = "i"


def _coords_of_logical(p):
    z = p // 8
    r8 = lax.rem(p, 8)
    y = r8 // 2
    j = lax.rem(r8, 2)
    x = jnp.where(lax.rem(y, 2) == 0, j, 1 - j)
    return x, y, z


def _logical_of_coords(x, y, z):
    xl = jnp.where(lax.rem(y, 2) == 0, x, 1 - x)
    return z * 8 + y * 2 + xl


def _rank_of_logical(p):
    x, y, z = _coords_of_logical(p)
    k = z * 4 + jnp.where(lax.rem(z, 2) == 0, y, 3 - y)
    return jnp.where(x == 0, k, 31 - k)


def _logical_of_rank(r):
    r = lax.rem(r + 2 * N_DEV, N_DEV)
    k = jnp.where(r < 16, r, 31 - r)
    x = jnp.where(r < 16, 0, 1)
    z = k // 4
    y4 = lax.rem(k, 4)
    y = jnp.where(lax.rem(z, 2) == 0, y4, 3 - y4)
    return _logical_of_coords(x, y, z)


def kernel(x, w_mat):
    m, k_shard = x.shape
    _, n = w_mat.shape
    blk = m // N_DEV
    nh = n // 2
    nc = nh // 2

    def body(x_ref, w_ref, out_ref, comm_r, comm_l, acc_ref,
             send_sems_r, recv_sems_r, send_sems_l, recv_sems_l,
             amax_send_ref, amax_all_ref, amax_send_sems, amax_recv_sems):
        p = lax.axis_index(AXIS)
        rk = _rank_of_logical(p)
        right = _logical_of_rank(rk + 1)
        left = _logical_of_rank(rk - 1)

        barrier_sem = pltpu.get_barrier_semaphore()
        pl.semaphore_signal(barrier_sem, inc=1, device_id=(left,),
                            device_id_type=pl.DeviceIdType.MESH)
        pl.semaphore_signal(barrier_sem, inc=1, device_id=(right,),
                            device_id_type=pl.DeviceIdType.MESH)
        pl.semaphore_wait(barrier_sem, 2)

        def contrib_r(dest_rank):
            b = _logical_of_rank(dest_rank)
            xb = x_ref[pl.ds(b * blk, blk), :]
            return jnp.dot(xb, w_ref[:, :nh], preferred_element_type=jnp.float32)

        def contrib_l(dest_rank):
            b = _logical_of_rank(dest_rank)
            xb = x_ref[pl.ds(b * blk, blk), :]
            return jnp.dot(xb, w_ref[:, nh:], preferred_element_type=jnp.float32)

        def rdma_hop(comm, send_sems, recv_sems, dev, h, c):
            return pltpu.make_async_remote_copy(
                src_ref=comm.at[h % 2, c],
                dst_ref=comm.at[(h + 1) % 2, c],
                send_sem=send_sems.at[h % 2, c],
                recv_sem=recv_sems.at[(h + 1) % 2, c],
                device_id=(dev,),
                device_id_type=pl.DeviceIdType.MESH,
            )

        first_r = contrib_r(rk - 1)
        first_l = contrib_l(rk + 1)
        comm_r[0, 0] = first_r[:, :nc]
        comm_r[0, 1] = first_r[:, nc:]
        comm_l[0, 0] = first_l[:, :nc]
        comm_l[0, 1] = first_l[:, nc:]
        for c in range(2):
            rdma_hop(comm_r, send_sems_r, recv_sems_r, right, 0, c).start()
            rdma_hop(comm_l, send_sems_l, recv_sems_l, left, 0, c).start()

        for s in range(N_DEV - 1):
            nxt_r = contrib_r(rk - s - 2)
            nxt_l = contrib_l(rk + s + 2)
            if s < N_DEV - 2:
                for c in range(2):
                    for comm, ssem, rsem, dev, nxt in (
                        (comm_r, send_sems_r, recv_sems_r, right, nxt_r),
                        (comm_l, send_sems_l, recv_sems_l, left, nxt_l),
                    ):
                        rdma_hop(comm, ssem, rsem, dev, s, c).wait_recv()
                        if s >= 1:
                            rdma_hop(comm, ssem, rsem, dev, s - 1, c).wait_send()
                        comm[(s + 1) % 2, c] = (
                            comm[(s + 1) % 2, c] + nxt[:, c * nc:(c + 1) * nc]
                        )
                        rdma_hop(comm, ssem, rsem, dev, s + 1, c).start()
            else:
                for c in range(2):
                    rdma_hop(comm_r, send_sems_r, recv_sems_r, right, s, c).wait_recv()
                    acc_ref[:, c * nc:(c + 1) * nc] = (
                        comm_r[(s + 1) % 2, c] + nxt_r[:, c * nc:(c + 1) * nc]
                    )
                    rdma_hop(comm_l, send_sems_l, recv_sems_l, left, s, c).wait_recv()
                    acc_ref[:, nh + c * nc:nh + (c + 1) * nc] = (
                        comm_l[(s + 1) % 2, c] + nxt_l[:, c * nc:(c + 1) * nc]
                    )

        for c in range(2):
            for h in (N_DEV - 3, N_DEV - 2):
                rdma_hop(comm_r, send_sems_r, recv_sems_r, right, h, c).wait_send()
                rdma_hop(comm_l, send_sems_l, recv_sems_l, left, h, c).wait_send()

        y = acc_ref[...]
        amax = jnp.max(jnp.abs(y))
        amax_send_ref[...] = jnp.full((8, 128), amax, jnp.float32)
        amax_all_ref[pl.ds(p, 1)] = amax_send_ref[...].reshape(1, 8, 128)
        for t in range(N_DEV):
            @pl.when(t != p)
            def _():
                pltpu.make_async_remote_copy(
                    src_ref=amax_send_ref,
                    dst_ref=amax_all_ref.at[p],
                    send_sem=amax_send_sems.at[t],
                    recv_sem=amax_recv_sems.at[p],
                    device_id=(t,),
                    device_id_type=pl.DeviceIdType.MESH,
                ).start()
        for t in range(N_DEV):
            @pl.when(t != p)
            def _():
                rx = pltpu.make_async_remote_copy(
                    src_ref=amax_send_ref,
                    dst_ref=amax_all_ref.at[t],
                    send_sem=amax_send_sems.at[t],
                    recv_sem=amax_recv_sems.at[t],
                    device_id=(t,),
                    device_id_type=pl.DeviceIdType.MESH,
                )
                rx.wait_recv()
                tx = pltpu.make_async_remote_copy(
                    src_ref=amax_send_ref,
                    dst_ref=amax_all_ref.at[p],
                    send_sem=amax_send_sems.at[t],
                    recv_sem=amax_recv_sems.at[p],
                    device_id=(t,),
                    device_id_type=pl.DeviceIdType.MESH,
                )
                tx.wait_send()
        amax_g = jnp.max(amax_all_ref[:, 0, 0])

        scale = amax_g / 127.0
        q = jnp.clip(jnp.round(y / scale), -127.0, 127.0)
        out_ref[...] = q * scale

    return pl.pallas_call(
        body,
        out_shape=jax.ShapeDtypeStruct((blk, n), jnp.float32),
        in_specs=[
            pl.BlockSpec(memory_space=pltpu.VMEM),
            pl.BlockSpec(memory_space=pltpu.VMEM),
        ],
        out_specs=pl.BlockSpec(memory_space=pltpu.VMEM),
        scratch_shapes=[
            pltpu.VMEM((2, 2, blk, nc), jnp.float32),
            pltpu.VMEM((2, 2, blk, nc), jnp.float32),
            pltpu.VMEM((blk, n), jnp.float32),
            pltpu.SemaphoreType.DMA((2, 2)),
            pltpu.SemaphoreType.DMA((2, 2)),
            pltpu.SemaphoreType.DMA((2, 2)),
            pltpu.SemaphoreType.DMA((2, 2)),
            pltpu.VMEM((8, 128), jnp.float32),
            pltpu.VMEM((N_DEV, 8, 128), jnp.float32),
            pltpu.SemaphoreType.DMA((N_DEV,)),
            pltpu.SemaphoreType.DMA((N_DEV,)),
        ],
        compiler_params=pltpu.CompilerParams(collective_id=0),
    )(x, w_mat)


# device time: 150962 ns/iter; 2.8446x vs baseline; 1.2684x over previous
import jax
import jax.numpy as jnp
import numpy as np
from jax import lax
from jax.experimental import pallas as pl
from jax.experimental.pallas import tpu as pltpu

N_DEV = 32
AXIS = "i"

_CYCLES = np.array([
    [0, 1, 2, 3, 4, 5, 6, 7, 15, 14, 22, 30, 31, 23, 20, 12,
     13, 21, 29, 28, 27, 24, 25, 26, 18, 10, 9, 17, 16, 19, 11, 8],
    [0, 3, 2, 5, 4, 7, 6, 14, 15, 23, 22, 21, 13, 10, 11, 12,
     20, 28, 31, 30, 29, 26, 27, 19, 18, 17, 25, 24, 16, 8, 9, 1],
    [0, 8, 11, 19, 16, 24, 27, 26, 25, 17, 18, 21, 20, 23, 31, 28,
     29, 30, 22, 14, 6, 5, 2, 1, 9, 10, 13, 12, 15, 7, 4, 3],
], dtype=np.int32)

_SUCC = np.zeros((N_DEV, 3), np.int32)
_PRED = np.zeros((N_DEV, 3), np.int32)
_DEST = np.zeros((N_DEV, 3, N_DEV), np.int32)
for _r in range(3):
    _cyc = _CYCLES[_r]
    _rank = {int(_cyc[i]): i for i in range(N_DEV)}
    for _p in range(N_DEV):
        _k = _rank[_p]
        _SUCC[_p, _r] = _cyc[(_k + 1) % N_DEV]
        _PRED[_p, _r] = _cyc[(_k - 1) % N_DEV]
        for _i in range(N_DEV):
            _DEST[_p, _r, _i] = _cyc[(_k - _i - 1) % N_DEV]
        assert _DEST[_p, _r, N_DEV - 1] == _p

_COL_OFF = (0, 768, 1408)
_COL_W = (768, 640, 640)


def kernel(x, w_mat):
    m, k_shard = x.shape
    _, n = w_mat.shape
    blk = m // N_DEV

    p = lax.axis_index(AXIS)
    succ = jnp.take(jnp.asarray(_SUCC), p, axis=0)
    pred = jnp.take(jnp.asarray(_PRED), p, axis=0)
    dest = jnp.take(jnp.asarray(_DEST), p, axis=0)

    def body(x_ref, w_ref, succ_ref, pred_ref, dest_ref, out_ref,
             comm0, comm1, comm2, acc_ref,
             ss0, rs0, ss1, rs1, ss2, rs2,
             amax_send_ref, amax_all_ref, amax_send_sems, amax_recv_sems):
        my_p = lax.axis_index(AXIS)

        rings = [
            (comm0, ss0, rs0, succ_ref[0], _COL_OFF[0], _COL_W[0] // 2),
            (comm1, ss1, rs1, succ_ref[1], _COL_OFF[1], _COL_W[1] // 2),
            (comm2, ss2, rs2, succ_ref[2], _COL_OFF[2], _COL_W[2] // 2),
        ]

        barrier_sem = pltpu.get_barrier_semaphore()
        for r in range(3):
            pl.semaphore_signal(barrier_sem, inc=1, device_id=(pred_ref[r],),
                                device_id_type=pl.DeviceIdType.MESH)
        pl.semaphore_wait(barrier_sem, 3)

        def contrib(r, i, off, w):
            b = dest_ref[r, i]
            xb = x_ref[pl.ds(b * blk, blk), :]
            return jnp.dot(xb, w_ref[:, off:off + w],
                           preferred_element_type=jnp.float32)

        def rdma_hop(comm, ssem, rsem, dev, h, c):
            return pltpu.make_async_remote_copy(
                src_ref=comm.at[h % 2, c],
                dst_ref=comm.at[(h + 1) % 2, c],
                send_sem=ssem.at[h % 2, c],
                recv_sem=rsem.at[(h + 1) % 2, c],
                device_id=(dev,),
                device_id_type=pl.DeviceIdType.MESH,
            )

        for r, (comm, ssem, rsem, dev, off, cw) in enumerate(rings):
            first = contrib(r, 0, off, 2 * cw)
            comm[0, 0] = first[:, :cw]
            comm[0, 1] = first[:, cw:]
            for c in range(2):
                rdma_hop(comm, ssem, rsem, dev, 0, c).start()

        for s in range(N_DEV - 1):
            nxt = [contrib(r, s + 1, off, 2 * cw)
                   for r, (_, _, _, _, off, cw) in enumerate(rings)]
            if s < N_DEV - 2:
                for c in range(2):
                    for r, (comm, ssem, rsem, dev, off, cw) in enumerate(rings):
                        rdma_hop(comm, ssem, rsem, dev, s, c).wait_recv()
                        if s >= 1:
                            rdma_hop(comm, ssem, rsem, dev, s - 1, c).wait_send()
                        comm[(s + 1) % 2, c] = (
                            comm[(s + 1) % 2, c] + nxt[r][:, c * cw:(c + 1) * cw]
                        )
                        rdma_hop(comm, ssem, rsem, dev, s + 1, c).start()
            else:
                for r, (comm, ssem, rsem, dev, off, cw) in enumerate(rings):
                    for c in range(2):
                        rdma_hop(comm, ssem, rsem, dev, s, c).wait_recv()
                        acc_ref[:, off + c * cw:off + (c + 1) * cw] = (
                            comm[(s + 1) % 2, c] + nxt[r][:, c * cw:(c + 1) * cw]
                        )

        for comm, ssem, rsem, dev, off, cw in rings:
            for c in range(2):
                for h in (N_DEV - 3, N_DEV - 2):
                    rdma_hop(comm, ssem, rsem, dev, h, c).wait_send()

        y = acc_ref[...]
        amax = jnp.max(jnp.abs(y))
        amax_send_ref[...] = jnp.full((8, 128), amax, jnp.float32)
        amax_all_ref[pl.ds(my_p, 1)] = amax_send_ref[...].reshape(1, 8, 128)
        for t in range(N_DEV):
            @pl.when(t != my_p)
            def _():
                pltpu.make_async_remote_copy(
                    src_ref=amax_send_ref,
                    dst_ref=amax_all_ref.at[my_p],
                    send_sem=amax_send_sems.at[t],
                    recv_sem=amax_recv_sems.at[my_p],
                    device_id=(t,),
                    device_id_type=pl.DeviceIdType.MESH,
                ).start()
        for t in range(N_DEV):
            @pl.when(t != my_p)
            def _():
                rx = pltpu.make_async_remote_copy(
                    src_ref=amax_send_ref,
                    dst_ref=amax_all_ref.at[t],
                    send_sem=amax_send_sems.at[t],
                    recv_sem=amax_recv_sems.at[t],
                    device_id=(t,),
                    device_id_type=pl.DeviceIdType.MESH,
                )
                rx.wait_recv()
                tx = pltpu.make_async_remote_copy(
                    src_ref=amax_send_ref,
                    dst_ref=amax_all_ref.at[my_p],
                    send_sem=amax_send_sems.at[t],
                    recv_sem=amax_recv_sems.at[my_p],
                    device_id=(t,),
                    device_id_type=pl.DeviceIdType.MESH,
                )
                tx.wait_send()
        amax_g = jnp.max(amax_all_ref[:, 0, 0])

        scale = amax_g / 127.0
        q = jnp.clip(jnp.round(y / scale), -127.0, 127.0)
        out_ref[...] = q * scale

    return pl.pallas_call(
        body,
        out_shape=jax.ShapeDtypeStruct((blk, n), jnp.float32),
        in_specs=[
            pl.BlockSpec(memory_space=pltpu.VMEM),
            pl.BlockSpec(memory_space=pltpu.VMEM),
            pl.BlockSpec(memory_space=pltpu.SMEM),
            pl.BlockSpec(memory_space=pltpu.SMEM),
            pl.BlockSpec(memory_space=pltpu.SMEM),
        ],
        out_specs=pl.BlockSpec(memory_space=pltpu.VMEM),
        scratch_shapes=[
            pltpu.VMEM((2, 2, blk, _COL_W[0] // 2), jnp.float32),
            pltpu.VMEM((2, 2, blk, _COL_W[1] // 2), jnp.float32),
            pltpu.VMEM((2, 2, blk, _COL_W[2] // 2), jnp.float32),
            pltpu.VMEM((blk, n), jnp.float32),
            pltpu.SemaphoreType.DMA((2, 2)),
            pltpu.SemaphoreType.DMA((2, 2)),
            pltpu.SemaphoreType.DMA((2, 2)),
            pltpu.SemaphoreType.DMA((2, 2)),
            pltpu.SemaphoreType.DMA((2, 2)),
            pltpu.SemaphoreType.DMA((2, 2)),
            pltpu.VMEM((8, 128), jnp.float32),
            pltpu.VMEM((N_DEV, 8, 128), jnp.float32),
            pltpu.SemaphoreType.DMA((N_DEV,)),
            pltpu.SemaphoreType.DMA((N_DEV,)),
        ],
        compiler_params=pltpu.CompilerParams(collective_id=0),
    )(x, w_mat, succ, pred, dest)


# device time: 150900 ns/iter; 2.8458x vs baseline; 1.0004x over previous
import jax
import jax.numpy as jnp
import numpy as np
from jax import lax
from jax.experimental import pallas as pl
from jax.experimental.pallas import tpu as pltpu

N_DEV = 32
AXIS = "i"

_CYCLES = np.array([
    [0, 1, 2, 3, 4, 5, 6, 7, 15, 14, 22, 30, 31, 23, 20, 12,
     13, 21, 29, 28, 27, 24, 25, 26, 18, 10, 9, 17, 16, 19, 11, 8],
    [0, 3, 2, 5, 4, 7, 6, 14, 15, 23, 22, 21, 13, 10, 11, 12,
     20, 28, 31, 30, 29, 26, 27, 19, 18, 17, 25, 24, 16, 8, 9, 1],
    [0, 8, 11, 19, 16, 24, 27, 26, 25, 17, 18, 21, 20, 23, 31, 28,
     29, 30, 22, 14, 6, 5, 2, 1, 9, 10, 13, 12, 15, 7, 4, 3],
], dtype=np.int32)

_SUCC = np.zeros((N_DEV, 3), np.int32)
_PRED = np.zeros((N_DEV, 3), np.int32)
_DEST = np.zeros((N_DEV, 3, N_DEV), np.int32)
for _r in range(3):
    _cyc = _CYCLES[_r]
    _rank = {int(_cyc[i]): i for i in range(N_DEV)}
    for _p in range(N_DEV):
        _k = _rank[_p]
        _SUCC[_p, _r] = _cyc[(_k + 1) % N_DEV]
        _PRED[_p, _r] = _cyc[(_k - 1) % N_DEV]
        for _i in range(N_DEV):
            _DEST[_p, _r, _i] = _cyc[(_k - _i - 1) % N_DEV]
        assert _DEST[_p, _r, N_DEV - 1] == _p

_COL_OFF = (0, 768, 1408)
_COL_W = (768, 640, 640)


def kernel(x, w_mat):
    m, k_shard = x.shape
    _, n = w_mat.shape
    blk = m // N_DEV

    p = lax.axis_index(AXIS)
    succ = jnp.take(jnp.asarray(_SUCC), p, axis=0)
    pred = jnp.take(jnp.asarray(_PRED), p, axis=0)
    dest = jnp.take(jnp.asarray(_DEST), p, axis=0)

    def body(x_ref, w_ref, succ_ref, pred_ref, dest_ref, out_ref,
             comm0, comm1, comm2, acc_ref,
             ss0, rs0, ss1, rs1, ss2, rs2,
             amax_send_ref, amax_all_ref, amax_send_sems, amax_recv_sems):
        my_p = lax.axis_index(AXIS)

        rings = [
            (comm0, ss0, rs0, succ_ref[0], _COL_OFF[0], _COL_W[0] // 2),
            (comm1, ss1, rs1, succ_ref[1], _COL_OFF[1], _COL_W[1] // 2),
            (comm2, ss2, rs2, succ_ref[2], _COL_OFF[2], _COL_W[2] // 2),
        ]

        barrier_sem = pltpu.get_barrier_semaphore()
        for r in range(3):
            pl.semaphore_signal(barrier_sem, inc=1, device_id=(pred_ref[r],),
                                device_id_type=pl.DeviceIdType.MESH)
        pl.semaphore_wait(barrier_sem, 3)

        def contrib(r, i, off, w):
            b = dest_ref[r, i]
            xb = x_ref[pl.ds(b * blk, blk), :]
            return jnp.dot(xb, w_ref[:, off:off + w],
                           preferred_element_type=jnp.float32)

        def rdma_hop(comm, ssem, rsem, dev, par, c):
            return pltpu.make_async_remote_copy(
                src_ref=comm.at[par, c],
                dst_ref=comm.at[1 - par, c],
                send_sem=ssem.at[par, c],
                recv_sem=rsem.at[1 - par, c],
                device_id=(dev,),
                device_id_type=pl.DeviceIdType.MESH,
            )

        for r, (comm, ssem, rsem, dev, off, cw) in enumerate(rings):
            first = contrib(r, 0, off, 2 * cw)
            comm[0, 0] = first[:, :cw]
            comm[0, 1] = first[:, cw:]
            for c in range(2):
                rdma_hop(comm, ssem, rsem, dev, 0, c).start()

        def ring_iter(par, dest_i, first_iter):
            nxt = [contrib(r, dest_i, off, 2 * cw)
                   for r, (_, _, _, _, off, cw) in enumerate(rings)]
            for c in range(2):
                for r, (comm, ssem, rsem, dev, off, cw) in enumerate(rings):
                    rdma_hop(comm, ssem, rsem, dev, par, c).wait_recv()
                    if not first_iter:
                        rdma_hop(comm, ssem, rsem, dev, 1 - par, c).wait_send()
                    comm[1 - par, c] = (
                        comm[1 - par, c] + nxt[r][:, c * cw:(c + 1) * cw]
                    )
                    rdma_hop(comm, ssem, rsem, dev, 1 - par, c).start()

        ring_iter(0, 1, True)

        def pair_body(t, carry):
            ring_iter(1, 2 + 2 * t, False)
            ring_iter(0, 3 + 2 * t, False)
            return carry

        lax.fori_loop(0, (N_DEV - 4) // 2, pair_body, 0)
        ring_iter(1, N_DEV - 2, False)

        nxt = [contrib(r, N_DEV - 1, off, 2 * cw)
               for r, (_, _, _, _, off, cw) in enumerate(rings)]
        for r, (comm, ssem, rsem, dev, off, cw) in enumerate(rings):
            for c in range(2):
                rdma_hop(comm, ssem, rsem, dev, 0, c).wait_recv()
                acc_ref[:, off + c * cw:off + (c + 1) * cw] = (
                    comm[1, c] + nxt[r][:, c * cw:(c + 1) * cw]
                )

        for comm, ssem, rsem, dev, off, cw in rings:
            for c in range(2):
                for par in (1, 0):
                    rdma_hop(comm, ssem, rsem, dev, par, c).wait_send()

        y = acc_ref[...]
        amax = jnp.max(jnp.abs(y))
        amax_send_ref[...] = jnp.full((8, 128), amax, jnp.float32)
        amax_all_ref[pl.ds(my_p, 1)] = amax_send_ref[...].reshape(1, 8, 128)
        for t in range(N_DEV):
            @pl.when(t != my_p)
            def _():
                pltpu.make_async_remote_copy(
                    src_ref=amax_send_ref,
                    dst_ref=amax_all_ref.at[my_p],
                    send_sem=amax_send_sems.at[t],
                    recv_sem=amax_recv_sems.at[my_p],
                    device_id=(t,),
                    device_id_type=pl.DeviceIdType.MESH,
                ).start()
        for t in range(N_DEV):
            @pl.when(t != my_p)
            def _():
                rx = pltpu.make_async_remote_copy(
                    src_ref=amax_send_ref,
                    dst_ref=amax_all_ref.at[t],
                    send_sem=amax_send_sems.at[t],
                    recv_sem=amax_recv_sems.at[t],
                    device_id=(t,),
                    device_id_type=pl.DeviceIdType.MESH,
                )
                rx.wait_recv()
                tx = pltpu.make_async_remote_copy(
                    src_ref=amax_send_ref,
                    dst_ref=amax_all_ref.at[my_p],
                    send_sem=amax_send_sems.at[t],
                    recv_sem=amax_recv_sems.at[my_p],
                    device_id=(t,),
                    device_id_type=pl.DeviceIdType.MESH,
                )
                tx.wait_send()
        amax_g = jnp.max(amax_all_ref[:, 0, 0])

        scale = amax_g / 127.0
        q = jnp.clip(jnp.round(y / scale), -127.0, 127.0)
        out_ref[...] = q * scale

    return pl.pallas_call(
        body,
        out_shape=jax.ShapeDtypeStruct((blk, n), jnp.float32),
        in_specs=[
            pl.BlockSpec(memory_space=pltpu.VMEM),
            pl.BlockSpec(memory_space=pltpu.VMEM),
            pl.BlockSpec(memory_space=pltpu.SMEM),
            pl.BlockSpec(memory_space=pltpu.SMEM),
            pl.BlockSpec(memory_space=pltpu.SMEM),
        ],
        out_specs=pl.BlockSpec(memory_space=pltpu.VMEM),
        scratch_shapes=[
            pltpu.VMEM((2, 2, blk, _COL_W[0] // 2), jnp.float32),
            pltpu.VMEM((2, 2, blk, _COL_W[1] // 2), jnp.float32),
            pltpu.VMEM((2, 2, blk, _COL_W[2] // 2), jnp.float32),
            pltpu.VMEM((blk, n), jnp.float32),
            pltpu.SemaphoreType.DMA((2, 2)),
            pltpu.SemaphoreType.DMA((2, 2)),
            pltpu.SemaphoreType.DMA((2, 2)),
            pltpu.SemaphoreType.DMA((2, 2)),
            pltpu.SemaphoreType.DMA((2, 2)),
            pltpu.SemaphoreType.DMA((2, 2)),
            pltpu.VMEM((8, 128), jnp.float32),
            pltpu.VMEM((N_DEV, 8, 128), jnp.float32),
            pltpu.SemaphoreType.DMA((N_DEV,)),
            pltpu.SemaphoreType.DMA((N_DEV,)),
        ],
        compiler_params=pltpu.CompilerParams(collective_id=0),
    )(x, w_mat, succ, pred, dest)
